# baseline (speedup 1.0000x reference)
import functools

import jax
import jax.numpy as jnp
from jax.experimental import pallas as pl

D_MODEL_K = 4096
N_EXPERTS_K = 64
K_TOP = 8
BLOCK_T = 1024
SUB_T = 256


def _gate_kernel(x_ref, w_ref, b_ref, idx_ref, nw_ref, probs_ref, mask_ref):
    x = x_ref[:]
    w = w_ref[:]
    logits = jax.lax.dot_general(
        x, w, (((1,), (1,)), ((), ())), preferred_element_type=jnp.float32
    )
    logits = logits + b_ref[:]

    m = jnp.max(logits, axis=1, keepdims=True)
    e = jnp.exp(logits - m)
    probs = e / jnp.sum(e, axis=1, keepdims=True)
    probs_ref[:] = probs

    for s in range(BLOCK_T // SUB_T):
        rows = slice(s * SUB_T, (s + 1) * SUB_T)
        p = probs[rows, :]
        lane = jax.lax.broadcasted_iota(jnp.int32, p.shape, 1).astype(
            jnp.float32
        )
        work = p
        vals = []
        idxs = []
        for _ in range(K_TOP):
            mx = jnp.max(work, axis=1, keepdims=True)
            cand = jnp.where(work == mx, lane, float(N_EXPERTS_K))
            amax = jnp.min(cand, axis=1, keepdims=True)
            vals.append(mx)
            idxs.append(amax)
            work = jnp.where(lane == amax, -1.0, work)

        mask_ref[rows, :] = jnp.where(work < 0.0, 1.0, 0.0)
        vals_cat = jnp.concatenate(vals, axis=1)
        idxs_cat = jnp.concatenate(idxs, axis=1)
        nw_ref[rows, :] = vals_cat / (
            jnp.sum(vals_cat, axis=1, keepdims=True) + 1e-9
        )
        idx_ref[rows, :] = idxs_cat.astype(jnp.int32)


@jax.jit
def kernel(x, W, b):
    n_tokens = x.shape[0]
    grid = (n_tokens // BLOCK_T,)
    b2 = b.reshape(1, N_EXPERTS_K)
    out_shapes = (
        jax.ShapeDtypeStruct((n_tokens, K_TOP), jnp.int32),
        jax.ShapeDtypeStruct((n_tokens, K_TOP), jnp.float32),
        jax.ShapeDtypeStruct((n_tokens, N_EXPERTS_K), jnp.float32),
        jax.ShapeDtypeStruct((n_tokens, N_EXPERTS_K), jnp.float32),
    )
    in_specs = [
        pl.BlockSpec((BLOCK_T, D_MODEL_K), lambda i: (i, 0)),
        pl.BlockSpec((N_EXPERTS_K, D_MODEL_K), lambda i: (0, 0)),
        pl.BlockSpec((1, N_EXPERTS_K), lambda i: (0, 0)),
    ]
    out_specs = (
        pl.BlockSpec((BLOCK_T, K_TOP), lambda i: (i, 0)),
        pl.BlockSpec((BLOCK_T, K_TOP), lambda i: (i, 0)),
        pl.BlockSpec((BLOCK_T, N_EXPERTS_K), lambda i: (i, 0)),
        pl.BlockSpec((BLOCK_T, N_EXPERTS_K), lambda i: (i, 0)),
    )
    topk_idx, norm_weights, gate_probs, hard_mask = pl.pallas_call(
        _gate_kernel,
        grid=grid,
        in_specs=in_specs,
        out_specs=out_specs,
        out_shape=out_shapes,
    )(x, W, b2)
    return (topk_idx, norm_weights, gate_probs, hard_mask)
